# baseline (speedup 1.0000x reference)
import functools

import jax
import jax.numpy as jnp
from jax import lax
from jax.experimental import pallas as pl
from jax.experimental.pallas import tpu as pltpu
from jax.experimental.pallas import tpu_sc as plsc

_NE = 8192
_CD = 64
_CIN = 768
_M = 9216
_MB = 2304
_KB = 1024


def _bdot(a, b, dims=(((1,), (0,)), ((), ()))):
    return lax.dot_general(a.astype(jnp.bfloat16), b.astype(jnp.bfloat16),
                           dims, preferred_element_type=jnp.float32)


def _dist_argmin_body(z_ref, emb_ref, wct_ref, bct_ref, wc_ref, bc_ref,
                      idx_ref, cb_ref, loss_ref, cn_ref, cbb_ref, acc_ref):
    i = pl.program_id(0)

    @pl.when(i == 0)
    def _init():
        cb = _bdot(emb_ref[...], wct_ref[...],
                   (((0,), (0,)), ((), ()))) + bct_ref[...]
        cbp = jnp.concatenate([cb, jnp.zeros_like(cb)], axis=1)
        cb_ref[...] = cbp
        cbb_ref[...] = cbp.astype(jnp.bfloat16)
        cn_ref[...] = jnp.sum(cb * cb, axis=1, keepdims=True)

    zc = _bdot(z_ref[...], wc_ref[...], (((1,), (1,)), ((), ()))) + bc_ref[...]
    zcp = jnp.concatenate([zc, jnp.zeros_like(zc)], axis=1)
    zcnb = (-2.0 * zcp).astype(jnp.bfloat16)

    R = 8
    iota_sub = lax.broadcasted_iota(jnp.int32, (R, _MB), 0)

    def mm(k):
        acc = lax.dot_general(cbb_ref[pl.ds(k * _KB, _KB), :], zcnb,
                              (((1,), (1,)), ((), ())),
                              preferred_element_type=jnp.float32)
        return acc + cn_ref[pl.ds(k * _KB, _KB), :]

    def scan(s, k, bmin, bidx):
        minv = s[:R, :]
        mini = jnp.zeros((R, _MB), jnp.int32)
        for r in range(1, _KB // R):
            sr = s[r * R:(r + 1) * R, :]
            c = sr < minv
            minv = jnp.where(c, sr, minv)
            mini = jnp.where(c, jnp.full((R, _MB), r * R, jnp.int32), mini)
        full_idx = mini + iota_sub + k * _KB
        take = minv < bmin
        return jnp.where(take, minv, bmin), jnp.where(take, full_idx, bidx)

    def chunk(k, carry):
        bmin, bidx, s = carry
        s_next = mm(k + 1)
        bmin, bidx = scan(s, k, bmin, bidx)
        return bmin, bidx, s_next

    nk = _NE // _KB
    init = (jnp.full((R, _MB), jnp.inf, jnp.float32),
            jnp.zeros((R, _MB), jnp.int32), mm(0))
    bmin, bidx, s_last = lax.fori_loop(0, nk - 1, chunk, init)
    v, ix = scan(s_last, nk - 1, bmin, bidx)
    h = R
    while h > 1:
        h //= 2
        av, bv = v[:h, :], v[h:2 * h, :]
        ai, bi = ix[:h, :], ix[h:2 * h, :]
        c = (bv < av) | ((bv == av) & (bi < ai))
        v = jnp.where(c, bv, av)
        ix = jnp.where(c, bi, ai)
    idx_ref[...] = ix[0:1, :][None]

    part = jnp.sum(zcp * zcp) + jnp.sum(v)

    @pl.when(i == 0)
    def _first():
        acc_ref[0, 0] = part

    @pl.when(i != 0)
    def _rest():
        acc_ref[0, 0] = acc_ref[0, 0] + part

    @pl.when(i == pl.num_programs(0) - 1)
    def _last():
        loss_ref[0, 0] = 3.0 * acc_ref[0, 0] / float(_M * _CD)


def _tc_dist_argmin(z2, emb, wct, bct2, wc, bc2):
    grid = _M // _MB
    return pl.pallas_call(
        _dist_argmin_body,
        grid=(grid,),
        in_specs=[
            pl.BlockSpec((_MB, _CIN), lambda i: (i, 0)),
            pl.BlockSpec((_CD, _NE), lambda i: (0, 0)),
            pl.BlockSpec((_CD, _CD), lambda i: (0, 0)),
            pl.BlockSpec((1, _CD), lambda i: (0, 0)),
            pl.BlockSpec((_CD, _CIN), lambda i: (0, 0)),
            pl.BlockSpec((1, _CD), lambda i: (0, 0)),
        ],
        out_specs=[
            pl.BlockSpec((1, 1, _MB), lambda i: (i, 0, 0)),
            pl.BlockSpec((_NE, 2 * _CD), lambda i: (0, 0)),
            pl.BlockSpec((1, 1), lambda i: (0, 0), memory_space=pltpu.SMEM),
        ],
        out_shape=[
            jax.ShapeDtypeStruct((_M // _MB, 1, _MB), jnp.int32),
            jax.ShapeDtypeStruct((_NE, 2 * _CD), jnp.float32),
            jax.ShapeDtypeStruct((1, 1), jnp.float32),
        ],
        scratch_shapes=[
            pltpu.VMEM((_NE, 1), jnp.float32),
            pltpu.VMEM((_NE, 2 * _CD), jnp.bfloat16),
            pltpu.SMEM((1, 1), jnp.float32),
        ],
    )(z2, emb, wct, bct2, wc, bc2)


def _expand_body(zq_ref, wx_ref, bx_ref, out_ref):
    out_ref[...] = _bdot(zq_ref[...], wx_ref[...]) + bx_ref[...]


_MB2 = 3072


def _tc_expand(zq, wxp, bx2):
    grid = _M // _MB2
    return pl.pallas_call(
        _expand_body,
        grid=(grid,),
        in_specs=[
            pl.BlockSpec((_MB2, 2 * _CD), lambda i: (i, 0)),
            pl.BlockSpec((2 * _CD, _CIN), lambda i: (0, 0)),
            pl.BlockSpec((1, _CIN), lambda i: (0, 0)),
        ],
        out_specs=pl.BlockSpec((_MB2, _CIN), lambda i: (i, 0)),
        out_shape=jax.ShapeDtypeStruct((_M, _CIN), jnp.float32),
    )(zq, wxp, bx2)


def _sc_gather(table, idx):
    info = plsc.get_sparse_core_info()
    nc, ns = info.num_cores, info.num_subcores
    nw = nc * ns
    bpw = _M // nw
    mesh = plsc.VectorSubcoreMesh(core_axis_name="c", subcore_axis_name="s")

    @functools.partial(
        pl.kernel, mesh=mesh,
        out_type=jax.ShapeDtypeStruct((_M, 2 * _CD), jnp.float32),
        scratch_types=[
            pltpu.VMEM((bpw,), jnp.int32),
            pltpu.VMEM((bpw // 2, 2 * _CD), jnp.float32),
            pltpu.VMEM((bpw // 2, 2 * _CD), jnp.float32),
            pltpu.SemaphoreType.DMA,
            pltpu.SemaphoreType.DMA,
        ],
    )
    def gather(table_hbm, idx_hbm, out_hbm, idx_v, rows_a, rows_b,
               sem_a, sem_b):
        wid = lax.axis_index("s") * nc + lax.axis_index("c")
        base = wid * bpw
        h = bpw // 2
        pltpu.sync_copy(idx_hbm.at[pl.ds(base, bpw)], idx_v)
        ca = pltpu.async_copy(table_hbm.at[idx_v.at[pl.ds(0, h)]],
                              rows_a, sem_a)
        cb_ = pltpu.async_copy(table_hbm.at[idx_v.at[pl.ds(h, h)]],
                               rows_b, sem_b)
        ca.wait()
        pltpu.sync_copy(rows_a, out_hbm.at[pl.ds(base, h)])
        cb_.wait()
        pltpu.sync_copy(rows_b, out_hbm.at[pl.ds(base + h, h)])

    return gather(table, idx)


def kernel(z, emb, W_ct, b_ct, W_c, b_c, W_x, b_x):
    B, T, CIN = z.shape
    z2 = z.reshape(-1, CIN)
    idx, cb, loss = _tc_dist_argmin(z2, emb.T, W_ct, b_ct.reshape(1, -1),
                                    W_c.T, b_c.reshape(1, -1))
    zq = _sc_gather(cb, idx.reshape(-1))
    wxp = jnp.concatenate([W_x, jnp.zeros_like(W_x)], axis=0)
    out2 = _tc_expand(zq, wxp, b_x.reshape(1, -1))
    return out2.reshape(B, T, CIN), loss[0, 0]


# speedup vs baseline: 1.8891x; 1.8891x over previous
import functools

import jax
import jax.numpy as jnp
from jax import lax
from jax.experimental import pallas as pl
from jax.experimental.pallas import tpu as pltpu
from jax.experimental.pallas import tpu_sc as plsc

_NE = 8192
_CD = 64
_CIN = 768
_M = 9216
_MB = 3072
_KB = 2048


def _bdot(a, b, dims=(((1,), (0,)), ((), ()))):
    return lax.dot_general(a.astype(jnp.bfloat16), b.astype(jnp.bfloat16),
                           dims, preferred_element_type=jnp.float32)


def _dist_argmin_body(z_ref, emb_ref, wct_ref, bct_ref, wc_ref, bc_ref,
                      idx_ref, cb_ref, loss_ref, cn_ref, cbb_ref, acc_ref):
    i = pl.program_id(0)

    @pl.when(i == 0)
    def _init():
        cb = _bdot(emb_ref[...], wct_ref[...],
                   (((0,), (0,)), ((), ()))) + bct_ref[...]
        cbp = jnp.concatenate([cb, jnp.zeros_like(cb)], axis=1)
        cb_ref[...] = cbp
        cbb_ref[...] = cbp.astype(jnp.bfloat16)
        cn_ref[...] = jnp.sum(cb * cb, axis=1, keepdims=True)

    zc = _bdot(z_ref[...], wc_ref[...], (((1,), (1,)), ((), ()))) + bc_ref[...]
    zcp = jnp.concatenate([zc, jnp.zeros_like(zc)], axis=1)
    zcnb = (-2.0 * zcp).astype(jnp.bfloat16)

    R = 8
    iota_sub = lax.broadcasted_iota(jnp.int32, (R, _MB), 0)

    def chunk(k, carry):
        bmin, bidx = carry
        acc = lax.dot_general(cbb_ref[pl.ds(k * _KB, _KB), :], zcnb,
                              (((1,), (1,)), ((), ())),
                              preferred_element_type=jnp.float32)
        s = acc + cn_ref[pl.ds(k * _KB, _KB), :]
        minv = s[:R, :]
        mini = jnp.zeros((R, _MB), jnp.int32)
        for r in range(1, _KB // R):
            sr = s[r * R:(r + 1) * R, :]
            c = sr < minv
            minv = jnp.where(c, sr, minv)
            mini = jnp.where(c, jnp.full((R, _MB), r * R, jnp.int32), mini)
        full_idx = mini + iota_sub + k * _KB
        take = minv < bmin
        return jnp.where(take, minv, bmin), jnp.where(take, full_idx, bidx)

    init = (jnp.full((R, _MB), jnp.inf, jnp.float32),
            jnp.zeros((R, _MB), jnp.int32))
    v, ix = lax.fori_loop(0, _NE // _KB, chunk, init)
    h = R
    while h > 1:
        h //= 2
        av, bv = v[:h, :], v[h:2 * h, :]
        ai, bi = ix[:h, :], ix[h:2 * h, :]
        c = (bv < av) | ((bv == av) & (bi < ai))
        v = jnp.where(c, bv, av)
        ix = jnp.where(c, bi, ai)
    idx_ref[...] = ix[0:1, :][None]

    part = jnp.sum(zcp * zcp) + jnp.sum(v)

    @pl.when(i == 0)
    def _first():
        acc_ref[0, 0] = part

    @pl.when(i != 0)
    def _rest():
        acc_ref[0, 0] = acc_ref[0, 0] + part

    @pl.when(i == pl.num_programs(0) - 1)
    def _last():
        loss_ref[0, 0] = 3.0 * acc_ref[0, 0] / float(_M * _CD)


def _tc_dist_argmin(z2, emb, wct, bct2, wc, bc2):
    grid = _M // _MB
    return pl.pallas_call(
        _dist_argmin_body,
        grid=(grid,),
        in_specs=[
            pl.BlockSpec((_MB, _CIN), lambda i: (i, 0)),
            pl.BlockSpec((_CD, _NE), lambda i: (0, 0)),
            pl.BlockSpec((_CD, _CD), lambda i: (0, 0)),
            pl.BlockSpec((1, _CD), lambda i: (0, 0)),
            pl.BlockSpec((_CD, _CIN), lambda i: (0, 0)),
            pl.BlockSpec((1, _CD), lambda i: (0, 0)),
        ],
        out_specs=[
            pl.BlockSpec((1, 1, _MB), lambda i: (i, 0, 0)),
            pl.BlockSpec((_NE, 2 * _CD), lambda i: (0, 0)),
            pl.BlockSpec((1, 1), lambda i: (0, 0), memory_space=pltpu.SMEM),
        ],
        out_shape=[
            jax.ShapeDtypeStruct((_M // _MB, 1, _MB), jnp.int32),
            jax.ShapeDtypeStruct((_NE, 2 * _CD), jnp.float32),
            jax.ShapeDtypeStruct((1, 1), jnp.float32),
        ],
        scratch_shapes=[
            pltpu.VMEM((_NE, 1), jnp.float32),
            pltpu.VMEM((_NE, 2 * _CD), jnp.bfloat16),
            pltpu.SMEM((1, 1), jnp.float32),
        ],
    )(z2, emb, wct, bct2, wc, bc2)


def _expand_body(zq_ref, wx_ref, bx_ref, out_ref):
    out_ref[...] = _bdot(zq_ref[...], wx_ref[...]) + bx_ref[...]


_MB2 = 3072


def _tc_expand(zq, wxp, bx2):
    grid = _M // _MB2
    return pl.pallas_call(
        _expand_body,
        grid=(grid,),
        in_specs=[
            pl.BlockSpec((_MB2, 2 * _CD), lambda i: (i, 0)),
            pl.BlockSpec((2 * _CD, _CIN), lambda i: (0, 0)),
            pl.BlockSpec((1, _CIN), lambda i: (0, 0)),
        ],
        out_specs=pl.BlockSpec((_MB2, _CIN), lambda i: (i, 0)),
        out_shape=jax.ShapeDtypeStruct((_M, _CIN), jnp.float32),
    )(zq, wxp, bx2)


def _sc_gather(table, idx):
    info = plsc.get_sparse_core_info()
    nc, ns = info.num_cores, info.num_subcores
    nw = nc * ns
    bpw = _M // nw
    mesh = plsc.VectorSubcoreMesh(core_axis_name="c", subcore_axis_name="s")

    @functools.partial(
        pl.kernel, mesh=mesh,
        out_type=jax.ShapeDtypeStruct((_M, 2 * _CD), jnp.float32),
        scratch_types=[
            pltpu.VMEM((bpw,), jnp.int32),
            pltpu.VMEM((bpw // 2, 2 * _CD), jnp.float32),
            pltpu.VMEM((bpw // 2, 2 * _CD), jnp.float32),
            pltpu.SemaphoreType.DMA,
            pltpu.SemaphoreType.DMA,
        ],
    )
    def gather(table_hbm, idx_hbm, out_hbm, idx_v, rows_a, rows_b,
               sem_a, sem_b):
        wid = lax.axis_index("s") * nc + lax.axis_index("c")
        base = wid * bpw
        h = bpw // 2
        pltpu.sync_copy(idx_hbm.at[pl.ds(base, bpw)], idx_v)
        ca = pltpu.async_copy(table_hbm.at[idx_v.at[pl.ds(0, h)]],
                              rows_a, sem_a)
        cb_ = pltpu.async_copy(table_hbm.at[idx_v.at[pl.ds(h, h)]],
                               rows_b, sem_b)
        ca.wait()
        pltpu.sync_copy(rows_a, out_hbm.at[pl.ds(base, h)])
        cb_.wait()
        pltpu.sync_copy(rows_b, out_hbm.at[pl.ds(base + h, h)])

    return gather(table, idx)


def kernel(z, emb, W_ct, b_ct, W_c, b_c, W_x, b_x):
    B, T, CIN = z.shape
    z2 = z.reshape(-1, CIN)
    idx, cb, loss = _tc_dist_argmin(z2, emb.T, W_ct, b_ct.reshape(1, -1),
                                    W_c.T, b_c.reshape(1, -1))
    zq = _sc_gather(cb, idx.reshape(-1))
    wxp = jnp.concatenate([W_x, jnp.zeros_like(W_x)], axis=0)
    out2 = _tc_expand(zq, wxp, b_x.reshape(1, -1))
    return out2.reshape(B, T, CIN), loss[0, 0]
